# baseline (speedup 1.0000x reference)
import jax
import jax.numpy as jnp
from jax.experimental import pallas as pl
from jax.experimental.pallas import tpu as pltpu

_N = 10000
_D_IN = 128
_D_H = 32
_D_OUT = 7
_D_CAT = _D_IN + _D_H
_BLOCK = 1000


def _dot(a, b):
    return jax.lax.dot_general(
        a, b, (((1,), (0,)), ((), ())), preferred_element_type=jnp.float32
    )


def _cell_body(x_ref, h_ref, wz_ref, bz_ref, wr_ref, br_ref, wh_ref, bh_ref,
               wl_ref, bl_ref, out_ref, hnew_ref):
    x = x_ref[...]
    h = h_ref[...]
    wz = wz_ref[0] + wz_ref[1]
    wr = wr_ref[0] + wr_ref[1]
    wh = wh_ref[0] + wh_ref[1]
    z = jax.nn.sigmoid(_dot(x, wz[:_D_IN]) + _dot(h, wz[_D_IN:]) + bz_ref[...])
    r = jax.nn.sigmoid(_dot(x, wr[:_D_IN]) + _dot(h, wr[_D_IN:]) + br_ref[...])
    ht = jnp.tanh(_dot(x, wh[:_D_IN]) + _dot(h * r, wh[_D_IN:]) + bh_ref[...])
    hn = z * h + (1.0 - z) * ht
    hnew_ref[...] = hn
    out_ref[...] = _dot(jnp.maximum(hn, 0.0), wl_ref[...]) + bl_ref[...]


def kernel(x, edge_index, edge_weight, h, W_z, b_z, W_r, b_r, W_h, b_h,
           W_lin, b_lin):
    del edge_index, edge_weight

    wz = W_z.reshape(2, _D_CAT, _D_H)
    wr = W_r.reshape(2, _D_CAT, _D_H)
    wh = W_h.reshape(2, _D_CAT, _D_H)
    bz = b_z.reshape(1, _D_H)
    br = b_r.reshape(1, _D_H)
    bh = b_h.reshape(1, _D_H)
    bl = b_lin.reshape(1, _D_OUT)

    grid = (_N // _BLOCK,)
    row_spec = lambda d: pl.BlockSpec((_BLOCK, d), lambda i: (i, 0))
    full2 = lambda s: pl.BlockSpec(s, lambda i: (0, 0))
    full3 = lambda s: pl.BlockSpec(s, lambda i: (0, 0, 0))

    out, hnew = pl.pallas_call(
        _cell_body,
        grid=grid,
        in_specs=[
            row_spec(_D_IN),
            row_spec(_D_H),
            full3((2, _D_CAT, _D_H)),
            full2((1, _D_H)),
            full3((2, _D_CAT, _D_H)),
            full2((1, _D_H)),
            full3((2, _D_CAT, _D_H)),
            full2((1, _D_H)),
            full2((_D_H, _D_OUT)),
            full2((1, _D_OUT)),
        ],
        out_specs=[
            row_spec(_D_OUT),
            row_spec(_D_H),
        ],
        out_shape=[
            jax.ShapeDtypeStruct((_N, _D_OUT), jnp.float32),
            jax.ShapeDtypeStruct((_N, _D_H), jnp.float32),
        ],
        compiler_params=pltpu.CompilerParams(
            dimension_semantics=("parallel",),
        ),
    )(x, h, wz, bz, wr, br, wh, bh, W_lin, bl)
    return out, hnew


# speedup vs baseline: 1.0487x; 1.0487x over previous
import jax
import jax.numpy as jnp
from jax.experimental import pallas as pl
from jax.experimental.pallas import tpu as pltpu

_N = 10000
_D_IN = 128
_D_H = 32
_D_OUT = 7
_D_CAT = _D_IN + _D_H
_BLOCK = 5000


def _dot(a, b):
    return jax.lax.dot_general(
        a, b, (((1,), (0,)), ((), ())), preferred_element_type=jnp.float32
    )


def _cell_body(x_ref, h_ref, wz_ref, bz_ref, wr_ref, br_ref, wh_ref, bh_ref,
               wl_ref, bl_ref, out_ref, hnew_ref):
    x = x_ref[...]
    h = h_ref[...]
    wz = wz_ref[0] + wz_ref[1]
    wr = wr_ref[0] + wr_ref[1]
    wh = wh_ref[0] + wh_ref[1]
    z = jax.nn.sigmoid(_dot(x, wz[:_D_IN]) + _dot(h, wz[_D_IN:]) + bz_ref[...])
    r = jax.nn.sigmoid(_dot(x, wr[:_D_IN]) + _dot(h, wr[_D_IN:]) + br_ref[...])
    ht = jnp.tanh(_dot(x, wh[:_D_IN]) + _dot(h * r, wh[_D_IN:]) + bh_ref[...])
    hn = z * h + (1.0 - z) * ht
    hnew_ref[...] = hn
    out_ref[...] = _dot(jnp.maximum(hn, 0.0), wl_ref[...]) + bl_ref[...]


def kernel(x, edge_index, edge_weight, h, W_z, b_z, W_r, b_r, W_h, b_h,
           W_lin, b_lin):
    del edge_index, edge_weight

    wz = W_z.reshape(2, _D_CAT, _D_H)
    wr = W_r.reshape(2, _D_CAT, _D_H)
    wh = W_h.reshape(2, _D_CAT, _D_H)
    bz = b_z.reshape(1, _D_H)
    br = b_r.reshape(1, _D_H)
    bh = b_h.reshape(1, _D_H)
    bl = b_lin.reshape(1, _D_OUT)

    grid = (_N // _BLOCK,)
    row_spec = lambda d: pl.BlockSpec((_BLOCK, d), lambda i: (i, 0))
    full2 = lambda s: pl.BlockSpec(s, lambda i: (0, 0))
    full3 = lambda s: pl.BlockSpec(s, lambda i: (0, 0, 0))

    out, hnew = pl.pallas_call(
        _cell_body,
        grid=grid,
        in_specs=[
            row_spec(_D_IN),
            row_spec(_D_H),
            full3((2, _D_CAT, _D_H)),
            full2((1, _D_H)),
            full3((2, _D_CAT, _D_H)),
            full2((1, _D_H)),
            full3((2, _D_CAT, _D_H)),
            full2((1, _D_H)),
            full2((_D_H, _D_OUT)),
            full2((1, _D_OUT)),
        ],
        out_specs=[
            row_spec(_D_OUT),
            row_spec(_D_H),
        ],
        out_shape=[
            jax.ShapeDtypeStruct((_N, _D_OUT), jnp.float32),
            jax.ShapeDtypeStruct((_N, _D_H), jnp.float32),
        ],
        compiler_params=pltpu.CompilerParams(
            dimension_semantics=("parallel",),
        ),
    )(x, h, wz, bz, wr, br, wh, bh, W_lin, bl)
    return out, hnew


# speedup vs baseline: 1.0984x; 1.0474x over previous
import jax
import jax.numpy as jnp
from jax.experimental import pallas as pl
from jax.experimental.pallas import tpu as pltpu

_N = 10000
_D_IN = 128
_D_H = 32
_D_OUT = 7
_D_CAT = _D_IN + _D_H
_BLOCK = 2000


def _dot(a, b):
    return jax.lax.dot_general(
        a, b, (((1,), (0,)), ((), ())), preferred_element_type=jnp.float32
    )


def _cell_body(x_ref, h_ref, wz_ref, bz_ref, wr_ref, br_ref, wh_ref, bh_ref,
               wl_ref, bl_ref, out_ref, hnew_ref):
    x = x_ref[...]
    h = h_ref[...]
    wz = wz_ref[0] + wz_ref[1]
    wr = wr_ref[0] + wr_ref[1]
    wh = wh_ref[0] + wh_ref[1]
    z = jax.nn.sigmoid(_dot(x, wz[:_D_IN]) + _dot(h, wz[_D_IN:]) + bz_ref[...])
    r = jax.nn.sigmoid(_dot(x, wr[:_D_IN]) + _dot(h, wr[_D_IN:]) + br_ref[...])
    ht = jnp.tanh(_dot(x, wh[:_D_IN]) + _dot(h * r, wh[_D_IN:]) + bh_ref[...])
    hn = z * h + (1.0 - z) * ht
    hnew_ref[...] = hn
    out_ref[...] = _dot(jnp.maximum(hn, 0.0), wl_ref[...]) + bl_ref[...]


def kernel(x, edge_index, edge_weight, h, W_z, b_z, W_r, b_r, W_h, b_h,
           W_lin, b_lin):
    del edge_index, edge_weight

    wz = W_z.reshape(2, _D_CAT, _D_H)
    wr = W_r.reshape(2, _D_CAT, _D_H)
    wh = W_h.reshape(2, _D_CAT, _D_H)
    bz = b_z.reshape(1, _D_H)
    br = b_r.reshape(1, _D_H)
    bh = b_h.reshape(1, _D_H)
    wl = jnp.pad(W_lin, ((0, 0), (0, 128 - _D_OUT)))
    bl = jnp.pad(b_lin.reshape(1, _D_OUT), ((0, 0), (0, 128 - _D_OUT)))

    grid = (_N // _BLOCK,)
    row_spec = lambda d: pl.BlockSpec((_BLOCK, d), lambda i: (i, 0))
    full2 = lambda s: pl.BlockSpec(s, lambda i: (0, 0))
    full3 = lambda s: pl.BlockSpec(s, lambda i: (0, 0, 0))

    out, hnew = pl.pallas_call(
        _cell_body,
        grid=grid,
        in_specs=[
            row_spec(_D_IN),
            row_spec(_D_H),
            full3((2, _D_CAT, _D_H)),
            full2((1, _D_H)),
            full3((2, _D_CAT, _D_H)),
            full2((1, _D_H)),
            full3((2, _D_CAT, _D_H)),
            full2((1, _D_H)),
            full2((_D_H, 128)),
            full2((1, 128)),
        ],
        out_specs=[
            row_spec(128),
            row_spec(_D_H),
        ],
        out_shape=[
            jax.ShapeDtypeStruct((_N, 128), jnp.float32),
            jax.ShapeDtypeStruct((_N, _D_H), jnp.float32),
        ],
        compiler_params=pltpu.CompilerParams(
            dimension_semantics=("parallel",),
        ),
    )(x, h, wz, bz, wr, br, wh, bh, wl, bl)
    return out[:, :_D_OUT], hnew


# speedup vs baseline: 1.1018x; 1.0031x over previous
import jax
import jax.numpy as jnp
from jax.experimental import pallas as pl
from jax.experimental.pallas import tpu as pltpu

_N = 10000
_D_IN = 128
_D_H = 32
_D_OUT = 7
_D_CAT = _D_IN + _D_H
_BLOCK = 10000


def _dot(a, b):
    return jax.lax.dot_general(
        a, b, (((1,), (0,)), ((), ())), preferred_element_type=jnp.float32
    )


def _cell_body(x_ref, h_ref, wz_ref, bz_ref, wr_ref, br_ref, wh_ref, bh_ref,
               wl_ref, bl_ref, out_ref, hnew_ref):
    x = x_ref[...]
    h = h_ref[...]
    wz = wz_ref[0] + wz_ref[1]
    wr = wr_ref[0] + wr_ref[1]
    wh = wh_ref[0] + wh_ref[1]
    z = jax.nn.sigmoid(_dot(x, wz[:_D_IN]) + _dot(h, wz[_D_IN:]) + bz_ref[...])
    r = jax.nn.sigmoid(_dot(x, wr[:_D_IN]) + _dot(h, wr[_D_IN:]) + br_ref[...])
    ht = jnp.tanh(_dot(x, wh[:_D_IN]) + _dot(h * r, wh[_D_IN:]) + bh_ref[...])
    hn = z * h + (1.0 - z) * ht
    hnew_ref[...] = hn
    out_ref[...] = _dot(jnp.maximum(hn, 0.0), wl_ref[...]) + bl_ref[...]


def kernel(x, edge_index, edge_weight, h, W_z, b_z, W_r, b_r, W_h, b_h,
           W_lin, b_lin):
    del edge_index, edge_weight

    wz = W_z.reshape(2, _D_CAT, _D_H)
    wr = W_r.reshape(2, _D_CAT, _D_H)
    wh = W_h.reshape(2, _D_CAT, _D_H)
    bz = b_z.reshape(1, _D_H)
    br = b_r.reshape(1, _D_H)
    bh = b_h.reshape(1, _D_H)
    wl = W_lin
    bl = b_lin.reshape(1, _D_OUT)

    grid = (_N // _BLOCK,)
    row_spec = lambda d: pl.BlockSpec((_BLOCK, d), lambda i: (i, 0))
    full2 = lambda s: pl.BlockSpec(s, lambda i: (0, 0))
    full3 = lambda s: pl.BlockSpec(s, lambda i: (0, 0, 0))

    out, hnew = pl.pallas_call(
        _cell_body,
        grid=grid,
        in_specs=[
            row_spec(_D_IN),
            row_spec(_D_H),
            full3((2, _D_CAT, _D_H)),
            full2((1, _D_H)),
            full3((2, _D_CAT, _D_H)),
            full2((1, _D_H)),
            full3((2, _D_CAT, _D_H)),
            full2((1, _D_H)),
            full2((_D_H, _D_OUT)),
            full2((1, _D_OUT)),
        ],
        out_specs=[
            row_spec(_D_OUT),
            row_spec(_D_H),
        ],
        out_shape=[
            jax.ShapeDtypeStruct((_N, _D_OUT), jnp.float32),
            jax.ShapeDtypeStruct((_N, _D_H), jnp.float32),
        ],
        compiler_params=pltpu.CompilerParams(
            dimension_semantics=("parallel",),
        ),
    )(x, h, wz, bz, wr, br, wh, bh, wl, bl)
    return out, hnew


# speedup vs baseline: 1.8339x; 1.6645x over previous
import jax
import jax.numpy as jnp
from jax.experimental import pallas as pl
from jax.experimental.pallas import tpu as pltpu

_N = 10000
_D_IN = 128
_D_H = 32
_D_OUT = 7
_D_CAT = _D_IN + _D_H
_BLK = 2560


def _cell_body(x_ref, hT_ref, wx_ref, wzr_ref, whh_ref, wl_ref, bz_ref,
               br_ref, bh_ref, bl_ref, outT_ref, HT_ref):
    x = x_ref[...]
    hT = hT_ref[...]

    gx = jax.lax.dot_general(
        wx_ref[...], x, (((0,), (1,)), ((), ())),
        preferred_element_type=jnp.float32)
    gzr = jax.lax.dot_general(
        wzr_ref[...], hT, (((0,), (0,)), ((), ())),
        preferred_element_type=jnp.float32)

    z = jax.nn.sigmoid(gx[0:32] + gzr[0:32] + bz_ref[...])
    r = jax.nn.sigmoid(gx[32:64] + gzr[32:64] + br_ref[...])
    hr = hT * r
    ghh = jax.lax.dot_general(
        whh_ref[...], hr, (((0,), (0,)), ((), ())),
        preferred_element_type=jnp.float32)
    ht = jnp.tanh(gx[64:96] + ghh + bh_ref[...])
    HT = z * hT + (1.0 - z) * ht
    HT_ref[...] = HT
    outT_ref[...] = jax.lax.dot_general(
        wl_ref[...], jnp.maximum(HT, 0.0), (((0,), (0,)), ((), ())),
        preferred_element_type=jnp.float32) + bl_ref[...]


def kernel(x, edge_index, edge_weight, h, W_z, b_z, W_r, b_r, W_h, b_h,
           W_lin, b_lin):
    del edge_index, edge_weight

    wz = W_z[0, 0] + W_z[1, 0]
    wr = W_r[0, 0] + W_r[1, 0]
    wh = W_h[0, 0] + W_h[1, 0]
    wx_all = jnp.concatenate([wz[:_D_IN], wr[:_D_IN], wh[:_D_IN]], axis=1)
    wzr = jnp.concatenate([wz[_D_IN:], wr[_D_IN:]], axis=1)
    whh = wh[_D_IN:]
    hT = h.T
    bz = b_z.reshape(_D_H, 1)
    br = b_r.reshape(_D_H, 1)
    bh = b_h.reshape(_D_H, 1)
    bl = b_lin.reshape(_D_OUT, 1)

    grid = (pl.cdiv(_N, _BLK),)
    col_spec = lambda d: pl.BlockSpec((d, _BLK), lambda i: (0, i))
    full2 = lambda s: pl.BlockSpec(s, lambda i: (0, 0))

    outT, HT = pl.pallas_call(
        _cell_body,
        grid=grid,
        in_specs=[
            pl.BlockSpec((_BLK, _D_IN), lambda i: (i, 0)),
            col_spec(_D_H),
            full2((_D_IN, 96)),
            full2((_D_H, 64)),
            full2((_D_H, _D_H)),
            full2((_D_H, _D_OUT)),
            full2((_D_H, 1)),
            full2((_D_H, 1)),
            full2((_D_H, 1)),
            full2((_D_OUT, 1)),
        ],
        out_specs=[
            col_spec(_D_OUT),
            col_spec(_D_H),
        ],
        out_shape=[
            jax.ShapeDtypeStruct((_D_OUT, _N), jnp.float32),
            jax.ShapeDtypeStruct((_D_H, _N), jnp.float32),
        ],
        compiler_params=pltpu.CompilerParams(
            dimension_semantics=("parallel",),
        ),
    )(x, hT, wx_all, wzr, whh, W_lin, bz, br, bh, bl)
    return outT.T, HT.T


# speedup vs baseline: 1.9009x; 1.0365x over previous
import jax
import jax.numpy as jnp
from jax.experimental import pallas as pl
from jax.experimental.pallas import tpu as pltpu

_N = 10000
_D_IN = 128
_D_H = 32
_D_OUT = 7
_D_CAT = _D_IN + _D_H
_BLK = 10000


def _cell_body(x_ref, hT_ref, wx_ref, wzr_ref, whh_ref, wl_ref, bz_ref,
               br_ref, bh_ref, bl_ref, outT_ref, HT_ref):
    x = x_ref[...]
    hT = hT_ref[...]

    gx = jax.lax.dot_general(
        wx_ref[...], x, (((0,), (1,)), ((), ())),
        preferred_element_type=jnp.float32)
    gzr = jax.lax.dot_general(
        wzr_ref[...], hT, (((0,), (0,)), ((), ())),
        preferred_element_type=jnp.float32)

    z = jax.nn.sigmoid(gx[0:32] + gzr[0:32] + bz_ref[...])
    r = jax.nn.sigmoid(gx[32:64] + gzr[32:64] + br_ref[...])
    hr = hT * r
    ghh = jax.lax.dot_general(
        whh_ref[...], hr, (((0,), (0,)), ((), ())),
        preferred_element_type=jnp.float32)
    ht = jnp.tanh(gx[64:96] + ghh + bh_ref[...])
    HT = z * hT + (1.0 - z) * ht
    HT_ref[...] = HT
    outT_ref[...] = jax.lax.dot_general(
        wl_ref[...], jnp.maximum(HT, 0.0), (((0,), (0,)), ((), ())),
        preferred_element_type=jnp.float32) + bl_ref[...]


def kernel(x, edge_index, edge_weight, h, W_z, b_z, W_r, b_r, W_h, b_h,
           W_lin, b_lin):
    del edge_index, edge_weight

    wz = W_z[0, 0] + W_z[1, 0]
    wr = W_r[0, 0] + W_r[1, 0]
    wh = W_h[0, 0] + W_h[1, 0]
    wx_all = jnp.concatenate([wz[:_D_IN], wr[:_D_IN], wh[:_D_IN]], axis=1)
    wzr = jnp.concatenate([wz[_D_IN:], wr[_D_IN:]], axis=1)
    whh = wh[_D_IN:]
    hT = h.T
    bz = b_z.reshape(_D_H, 1)
    br = b_r.reshape(_D_H, 1)
    bh = b_h.reshape(_D_H, 1)
    bl = b_lin.reshape(_D_OUT, 1)

    grid = (pl.cdiv(_N, _BLK),)
    col_spec = lambda d: pl.BlockSpec((d, _BLK), lambda i: (0, i))
    full2 = lambda s: pl.BlockSpec(s, lambda i: (0, 0))

    outT, HT = pl.pallas_call(
        _cell_body,
        grid=grid,
        in_specs=[
            pl.BlockSpec((_BLK, _D_IN), lambda i: (i, 0)),
            col_spec(_D_H),
            full2((_D_IN, 96)),
            full2((_D_H, 64)),
            full2((_D_H, _D_H)),
            full2((_D_H, _D_OUT)),
            full2((_D_H, 1)),
            full2((_D_H, 1)),
            full2((_D_H, 1)),
            full2((_D_OUT, 1)),
        ],
        out_specs=[
            col_spec(_D_OUT),
            col_spec(_D_H),
        ],
        out_shape=[
            jax.ShapeDtypeStruct((_D_OUT, _N), jnp.float32),
            jax.ShapeDtypeStruct((_D_H, _N), jnp.float32),
        ],
        compiler_params=pltpu.CompilerParams(
            dimension_semantics=("parallel",),
        ),
    )(x, hT, wx_all, wzr, whh, W_lin, bz, br, bh, bl)
    return outT.T, HT.T


# speedup vs baseline: 2.7174x; 1.4295x over previous
import jax
import jax.numpy as jnp
from jax.experimental import pallas as pl
from jax.experimental.pallas import tpu as pltpu

_N = 10000
_D_IN = 128
_D_H = 32
_D_OUT = 7
_D_CAT = _D_IN + _D_H
_BLK = 10000


def _cell_body(x_ref, hT_ref, w_ref, outT_ref, HT_ref):
    x = x_ref[...]
    hT = hT_ref[...]

    wx_all = w_ref[0:_D_IN, 0:96]
    wzr = w_ref[_D_IN:_D_CAT, 0:64]
    whh = w_ref[_D_IN:_D_CAT, 64:96]
    wl = w_ref[_D_IN:_D_CAT, 96:96 + _D_OUT]
    bz = w_ref[_D_CAT:_D_CAT + _D_H, 0:1]
    br = w_ref[_D_CAT:_D_CAT + _D_H, 1:2]
    bh = w_ref[_D_CAT:_D_CAT + _D_H, 2:3]
    bl = w_ref[_D_CAT:_D_CAT + _D_OUT, 3:4]

    gx = jax.lax.dot_general(
        wx_all, x, (((0,), (1,)), ((), ())),
        preferred_element_type=jnp.float32)
    gzr = jax.lax.dot_general(
        wzr, hT, (((0,), (0,)), ((), ())),
        preferred_element_type=jnp.float32)

    z = jax.nn.sigmoid(gx[0:32] + gzr[0:32] + bz)
    r = jax.nn.sigmoid(gx[32:64] + gzr[32:64] + br)
    hr = hT * r
    ghh = jax.lax.dot_general(
        whh, hr, (((0,), (0,)), ((), ())),
        preferred_element_type=jnp.float32)
    ht = jnp.tanh(gx[64:96] + ghh + bh)
    HT = z * hT + (1.0 - z) * ht
    HT_ref[...] = HT
    outT_ref[...] = jax.lax.dot_general(
        wl, jnp.maximum(HT, 0.0), (((0,), (0,)), ((), ())),
        preferred_element_type=jnp.float32) + bl


def kernel(x, edge_index, edge_weight, h, W_z, b_z, W_r, b_r, W_h, b_h,
           W_lin, b_lin):
    del edge_index, edge_weight

    wz = W_z[0, 0] + W_z[1, 0]
    wr = W_r[0, 0] + W_r[1, 0]
    wh = W_h[0, 0] + W_h[1, 0]
    wl = jnp.pad(W_lin, ((_D_IN, 0), (0, 0)))
    wtop = jnp.concatenate([wz, wr, wh, wl], axis=1)
    wtop = jnp.pad(wtop, ((0, 0), (0, 128 - 103)))
    bl = jnp.pad(b_lin, (0, _D_H - _D_OUT))
    brow = jnp.stack([b_z, b_r, b_h, bl], axis=1)
    brow = jnp.pad(brow, ((0, 0), (0, 124)))
    wpack = jnp.concatenate([wtop, brow], axis=0)

    hT = h.T

    grid = (pl.cdiv(_N, _BLK),)
    col_spec = lambda d: pl.BlockSpec((d, _BLK), lambda i: (0, i))

    outT, HT = pl.pallas_call(
        _cell_body,
        grid=grid,
        in_specs=[
            pl.BlockSpec((_BLK, _D_IN), lambda i: (i, 0)),
            col_spec(_D_H),
            pl.BlockSpec((192, 128), lambda i: (0, 0)),
        ],
        out_specs=[
            col_spec(_D_OUT),
            col_spec(_D_H),
        ],
        out_shape=[
            jax.ShapeDtypeStruct((_D_OUT, _N), jnp.float32),
            jax.ShapeDtypeStruct((_D_H, _N), jnp.float32),
        ],
        compiler_params=pltpu.CompilerParams(
            dimension_semantics=("parallel",),
        ),
    )(x, hT, wpack)
    return outT.T, HT.T


# speedup vs baseline: 2.8386x; 1.0446x over previous
import jax
import jax.numpy as jnp
from jax.experimental import pallas as pl
from jax.experimental.pallas import tpu as pltpu

_N = 10000
_D_IN = 128
_D_H = 32
_D_OUT = 7
_D_CAT = _D_IN + _D_H
_BLK = 5120


def _cell_body(x_ref, hT_ref, w_ref, outT_ref, HT_ref):
    x = x_ref[...]
    hT = hT_ref[...]

    wx_all = w_ref[0:_D_IN, 0:96]
    wzr = w_ref[_D_IN:_D_CAT, 0:64]
    whh = w_ref[_D_IN:_D_CAT, 64:96]
    wl = w_ref[_D_IN:_D_CAT, 96:96 + _D_OUT]
    bz = w_ref[_D_CAT:_D_CAT + _D_H, 0:1]
    br = w_ref[_D_CAT:_D_CAT + _D_H, 1:2]
    bh = w_ref[_D_CAT:_D_CAT + _D_H, 2:3]
    bl = w_ref[_D_CAT:_D_CAT + _D_OUT, 3:4]

    gx = jax.lax.dot_general(
        wx_all, x, (((0,), (1,)), ((), ())),
        preferred_element_type=jnp.float32)
    gzr = jax.lax.dot_general(
        wzr, hT, (((0,), (0,)), ((), ())),
        preferred_element_type=jnp.float32)

    z = jax.nn.sigmoid(gx[0:32] + gzr[0:32] + bz)
    r = jax.nn.sigmoid(gx[32:64] + gzr[32:64] + br)
    hr = hT * r
    ghh = jax.lax.dot_general(
        whh, hr, (((0,), (0,)), ((), ())),
        preferred_element_type=jnp.float32)
    ht = jnp.tanh(gx[64:96] + ghh + bh)
    HT = z * hT + (1.0 - z) * ht
    HT_ref[...] = HT
    outT_ref[...] = jax.lax.dot_general(
        wl, jnp.maximum(HT, 0.0), (((0,), (0,)), ((), ())),
        preferred_element_type=jnp.float32) + bl


def kernel(x, edge_index, edge_weight, h, W_z, b_z, W_r, b_r, W_h, b_h,
           W_lin, b_lin):
    del edge_index, edge_weight

    wz = W_z[0, 0] + W_z[1, 0]
    wr = W_r[0, 0] + W_r[1, 0]
    wh = W_h[0, 0] + W_h[1, 0]
    wl = jnp.pad(W_lin, ((_D_IN, 0), (0, 0)))
    wtop = jnp.concatenate([wz, wr, wh, wl], axis=1)
    wtop = jnp.pad(wtop, ((0, 0), (0, 128 - 103)))
    bl = jnp.pad(b_lin, (0, _D_H - _D_OUT))
    brow = jnp.stack([b_z, b_r, b_h, bl], axis=1)
    brow = jnp.pad(brow, ((0, 0), (0, 124)))
    wpack = jnp.concatenate([wtop, brow], axis=0)

    hT = h.T

    grid = (pl.cdiv(_N, _BLK),)
    col_spec = lambda d: pl.BlockSpec((d, _BLK), lambda i: (0, i))

    outT, HT = pl.pallas_call(
        _cell_body,
        grid=grid,
        in_specs=[
            pl.BlockSpec((_BLK, _D_IN), lambda i: (i, 0)),
            col_spec(_D_H),
            pl.BlockSpec((192, 128), lambda i: (0, 0)),
        ],
        out_specs=[
            col_spec(_D_OUT),
            col_spec(_D_H),
        ],
        out_shape=[
            jax.ShapeDtypeStruct((_D_OUT, _N), jnp.float32),
            jax.ShapeDtypeStruct((_D_H, _N), jnp.float32),
        ],
        compiler_params=pltpu.CompilerParams(
            dimension_semantics=("parallel",),
        ),
    )(x, hT, wpack)
    return outT.T, HT.T
